# baseline (speedup 1.0000x reference)
import jax
import jax.numpy as jnp
from jax import lax
from jax.experimental import pallas as pl
from jax.experimental.pallas import tpu as pltpu
from jax.experimental.pallas import tpu_sc as plsc

_B = 16384
_D = 128
_L = 16
_NCORES = 2
_NSUB = 16
_NW = _NCORES * _NSUB
_RPW = _B // _NW
_CH = 128
_W = 64
_CHS = (_W, _W, _CH, _CH, _CH)
_ST = (0, _W, 2 * _W, 2 * _W + _CH, 2 * _W + 2 * _CH)
_NCHUNK = len(_CHS)


def _body(x_hbm, y_hbm, fcw_hbm, fcb_hbm, emb_hbm, out_hbm,
          idxw_v, idx_v, x_v, e_v, out_v, par_v, fcb_v, tr_v,
          siw, si0, si1, sx0, sx1, se0, se1, so):
    cid = lax.axis_index("c")
    sid = lax.axis_index("s")
    base = (sid * _NCORES + cid) * _RPW

    semx = [sx0, sx1]
    seme = [se0, se1]

    def idx_ref(c):
        return idxw_v.at[c] if c < 2 else idx_v.at[c % 2]

    def issue_idx(c, sem):
        r0 = pl.multiple_of(base + _ST[c], _W)
        return pltpu.async_copy(y_hbm.at[pl.ds(r0, _CHS[c])], idx_ref(c), sem)

    def issue_data(c):
        b, n = c % 2, _CHS[c]
        r0 = pl.multiple_of(base + _ST[c], _W)
        hx = pltpu.async_copy(x_hbm.at[pl.ds(r0, n)],
                              x_v.at[b, pl.ds(0, n)], semx[b])
        he = pltpu.async_copy(emb_hbm.at[idx_ref(c)],
                              e_v.at[b, pl.ds(0, n)], seme[b])
        return hx, he

    pltpu.sync_copy(y_hbm.at[pl.ds(pl.multiple_of(base, _W), _W)],
                    idxw_v.at[0])
    pend = issue_data(0)
    hidx = {1: issue_idx(1, siw), 2: issue_idx(2, si0), 3: issue_idx(3, si1)}
    pltpu.sync_copy(fcw_hbm, par_v)
    pltpu.sync_copy(fcb_hbm, fcb_v)
    fcw = [par_v[0, pl.ds(_L * j, _L)] for j in range(_D // _L)]
    fcb_vec = plsc.load_gather(fcb_v, [jnp.zeros((_L,), jnp.int32)])
    gbase = lax.iota(jnp.int32, _L) * _L

    out_handles = []
    for c in range(_NCHUNK):
        b = c % 2
        hx, he = pend
        hx.wait()
        he.wait()
        if c + 1 < _NCHUNK:
            hidx[c + 1].wait()
            pend = issue_data(c + 1)
            if c == 2:
                hidx[4] = issue_idx(4, si0)

        def make_group(b, c):
            def group(g):
                tbase = pl.multiple_of(g * _L * _L, _L)
                for rr in range(_L):
                    r = g * _L + rr
                    acc = None
                    for j in range(_D // _L):
                        xv = x_v[b, r, pl.ds(_L * j, _L)]
                        ev = e_v[b, r, pl.ds(_L * j, _L)]
                        t = xv * (ev + fcw[j])
                        acc = t if acc is None else acc + t
                    tr_v[pl.ds(tbase + rr * _L, _L)] = acc
                outvec = fcb_vec
                for col in range(_L):
                    outvec = outvec + plsc.load_gather(
                        tr_v, [tbase + gbase + col])
                out_v[c, pl.ds(pl.multiple_of(g * _L, _L), _L)] = outvec
            return group

        plsc.parallel_loop(0, _CHS[c] // _L, step=1, unroll=2)(
            make_group(b, c))

        r0 = pl.multiple_of(base + _ST[c], _W)
        out_handles.append(
            pltpu.async_copy(out_v.at[c, pl.ds(0, _CHS[c])],
                             out_hbm.at[pl.ds(r0, _CHS[c])], so))
    for h in out_handles:
        h.wait()


_sc_call = pl.kernel(
    _body,
    out_type=jax.ShapeDtypeStruct((_B,), jnp.float32),
    mesh=plsc.VectorSubcoreMesh(
        core_axis_name="c", subcore_axis_name="s",
        num_cores=_NCORES, num_subcores=_NSUB),
    compiler_params=pltpu.CompilerParams(needs_layout_passes=False),
    scratch_types=[
        pltpu.VMEM((2, _W), jnp.int32),
        pltpu.VMEM((2, _CH), jnp.int32),
        pltpu.VMEM((2, _CH, _D), jnp.float32),
        pltpu.VMEM((2, _CH, _D), jnp.float32),
        pltpu.VMEM((_NCHUNK, _CH), jnp.float32),
        pltpu.VMEM((1, _D), jnp.float32),
        pltpu.VMEM((1,), jnp.float32),
        pltpu.VMEM((_CH // _L * _L * _L,), jnp.float32),
    ] + [pltpu.SemaphoreType.DMA] * 8,
)


def kernel(x, y, fc_w, fc_b, emb):
    return _sc_call(x, y.astype(jnp.int32), fc_w, fc_b, emb)


# speedup vs baseline: 1.0726x; 1.0726x over previous
import jax
import jax.numpy as jnp
from jax import lax
from jax.experimental import pallas as pl
from jax.experimental.pallas import tpu as pltpu
from jax.experimental.pallas import tpu_sc as plsc

_B = 16384
_D = 128
_L = 16
_NCORES = 2
_NSUB = 16
_NW = _NCORES * _NSUB
_RPW = _B // _NW
_CH = 128
_NCHUNK = _RPW // _CH
_NG = _CH // _L


def _body(x_hbm, y_hbm, fcw_hbm, fcb_hbm, emb_hbm, out_hbm,
          idx_v, x_v, e_v, out_v, par_v, fcb_v, tr_v,
          si0, si1, sx0, sx1, se0, se1, sxh, seh, so):
    cid = lax.axis_index("c")
    sid = lax.axis_index("s")
    base = (sid * _NCORES + cid) * _RPW

    semi = [si0, si1]
    semx = [sx0, sx1]
    seme = [se0, se1]

    def issue_idx(c):
        r0 = pl.multiple_of(base + c * _CH, _CH)
        return pltpu.async_copy(y_hbm.at[pl.ds(r0, _CH)], idx_v.at[c % 2],
                                semi[c % 2])

    def issue_data(c):
        b = c % 2
        r0 = pl.multiple_of(base + c * _CH, _CH)
        hx = pltpu.async_copy(x_hbm.at[pl.ds(r0, _CH)], x_v.at[b], semx[b])
        he = pltpu.async_copy(emb_hbm.at[idx_v.at[b]], e_v.at[b], seme[b])
        return hx, he

    pltpu.sync_copy(y_hbm.at[pl.ds(pl.multiple_of(base, _CH), _CH)],
                    idx_v.at[0])
    pend = issue_data(0)
    hidx = issue_idx(1)
    pltpu.sync_copy(fcw_hbm, par_v)
    pltpu.sync_copy(fcb_hbm, fcb_v)
    fcw = [par_v[0, pl.ds(_L * j, _L)] for j in range(_D // _L)]
    fcb_vec = plsc.load_gather(fcb_v, [jnp.zeros((_L,), jnp.int32)])
    gbase = lax.iota(jnp.int32, _L) * _L

    out_handles = []
    for c in range(_NCHUNK):
        b = c % 2
        hx, he = pend
        hx.wait()
        he.wait()
        if c + 1 < _NCHUNK:
            hidx.wait()
            pend = issue_data(c + 1)
            if c + 2 < _NCHUNK:
                hidx = issue_idx(c + 2)

        def make_group(b, c):
            def group(g):
                tbase = pl.multiple_of(g * _L * _L, _L)
                for rr in range(_L):
                    r = g * _L + rr
                    acc = None
                    for j in range(_D // _L):
                        xv = x_v[b, r, pl.ds(_L * j, _L)]
                        ev = e_v[b, r, pl.ds(_L * j, _L)]
                        t = xv * (ev + fcw[j])
                        acc = t if acc is None else acc + t
                    tr_v[pl.ds(tbase + rr * _L, _L)] = acc
                outvec = fcb_vec
                for col in range(_L):
                    outvec = outvec + plsc.load_gather(
                        tr_v, [tbase + gbase + col])
                out_v[c, pl.ds(pl.multiple_of(g * _L, _L), _L)] = outvec
            return group

        plsc.parallel_loop(0, _NG, step=1, unroll=2)(make_group(b, c))

        r0 = pl.multiple_of(base + c * _CH, _CH)
        out_handles.append(
            pltpu.async_copy(out_v.at[c], out_hbm.at[pl.ds(r0, _CH)], so))
    for h in out_handles:
        h.wait()


_sc_call = pl.kernel(
    _body,
    out_type=jax.ShapeDtypeStruct((_B,), jnp.float32),
    mesh=plsc.VectorSubcoreMesh(
        core_axis_name="c", subcore_axis_name="s",
        num_cores=_NCORES, num_subcores=_NSUB),
    compiler_params=pltpu.CompilerParams(needs_layout_passes=False),
    scratch_types=[
        pltpu.VMEM((2, _CH), jnp.int32),
        pltpu.VMEM((2, _CH, _D), jnp.float32),
        pltpu.VMEM((2, _CH, _D), jnp.float32),
        pltpu.VMEM((_NCHUNK, _CH), jnp.float32),
        pltpu.VMEM((1, _D), jnp.float32),
        pltpu.VMEM((1,), jnp.float32),
        pltpu.VMEM((_NG * _L * _L,), jnp.float32),
    ] + [pltpu.SemaphoreType.DMA] * 9,
)


def kernel(x, y, fc_w, fc_b, emb):
    return _sc_call(x, y.astype(jnp.int32), fc_w, fc_b, emb)


# speedup vs baseline: 1.0818x; 1.0086x over previous
import jax
import jax.numpy as jnp
from jax import lax
from jax.experimental import pallas as pl
from jax.experimental.pallas import tpu as pltpu
from jax.experimental.pallas import tpu_sc as plsc

_B = 16384
_D = 128
_L = 16
_NCORES = 2
_NSUB = 16
_NW = _NCORES * _NSUB
_RPW = _B // _NW
_CH = 128
_NCHUNK = _RPW // _CH
_NG = _CH // _L


def _body(x_hbm, y_hbm, fcw_hbm, fcb_hbm, emb_hbm, out_hbm,
          idx_v, x_v, e_v, out_v, par_v, fcb_v, tr_v,
          si0, si1, sx0, sx1, se0, se1, sxh, seh, so):
    cid = lax.axis_index("c")
    sid = lax.axis_index("s")
    base = (sid * _NCORES + cid) * _RPW

    semi = [si0, si1]
    semx = [sx0, sx1]
    seme = [se0, se1]

    def issue_idx(c):
        r0 = pl.multiple_of(base + c * _CH, _CH)
        return pltpu.async_copy(y_hbm.at[pl.ds(r0, _CH)], idx_v.at[c % 2],
                                semi[c % 2])

    def issue_data(c):
        b = c % 2
        r0 = pl.multiple_of(base + c * _CH, _CH)
        hx = pltpu.async_copy(x_hbm.at[pl.ds(r0, _CH)], x_v.at[b], semx[b])
        he = pltpu.async_copy(emb_hbm.at[idx_v.at[b]], e_v.at[b], seme[b])
        return hx, he

    pltpu.sync_copy(y_hbm.at[pl.ds(pl.multiple_of(base, _CH), _CH)],
                    idx_v.at[0])
    pend = issue_data(0)
    hidx = issue_idx(1)
    pltpu.sync_copy(fcw_hbm, par_v)
    pltpu.sync_copy(fcb_hbm, fcb_v)
    fcw = [par_v[0, pl.ds(_L * j, _L)] for j in range(_D // _L)]
    fcb_vec = plsc.load_gather(fcb_v, [jnp.zeros((_L,), jnp.int32)])
    gbase = lax.iota(jnp.int32, _L) * _L

    out_handles = []
    for c in range(_NCHUNK):
        b = c % 2
        hx, he = pend
        hx.wait()
        he.wait()
        if c + 1 < _NCHUNK:
            hidx.wait()
            pend = issue_data(c + 1)
            if c + 2 < _NCHUNK:
                hidx = issue_idx(c + 2)

        def make_group(b, c):
            def group(g):
                tbase = pl.multiple_of(g * _L * _L, _L)
                for rr in range(_L):
                    r = g * _L + rr
                    acc = None
                    for j in range(_D // _L):
                        xv = x_v[b, r, pl.ds(_L * j, _L)]
                        ev = e_v[b, r, pl.ds(_L * j, _L)]
                        t = xv * (ev + fcw[j])
                        acc = t if acc is None else acc + t
                    tr_v[pl.ds(tbase + rr * _L, _L)] = acc
                cols = [plsc.load_gather(tr_v, [tbase + gbase + col])
                        for col in range(_L)]
                while len(cols) > 1:
                    cols = [a + b2 for a, b2 in zip(cols[::2], cols[1::2])]
                out_v[c, pl.ds(pl.multiple_of(g * _L, _L), _L)] = (
                    cols[0] + fcb_vec)
            return group

        plsc.parallel_loop(0, _NG, step=1, unroll=2)(make_group(b, c))

        r0 = pl.multiple_of(base + c * _CH, _CH)
        out_handles.append(
            pltpu.async_copy(out_v.at[c], out_hbm.at[pl.ds(r0, _CH)], so))
    for h in out_handles:
        h.wait()


_sc_call = pl.kernel(
    _body,
    out_type=jax.ShapeDtypeStruct((_B,), jnp.float32),
    mesh=plsc.VectorSubcoreMesh(
        core_axis_name="c", subcore_axis_name="s",
        num_cores=_NCORES, num_subcores=_NSUB),
    compiler_params=pltpu.CompilerParams(needs_layout_passes=False),
    scratch_types=[
        pltpu.VMEM((2, _CH), jnp.int32),
        pltpu.VMEM((2, _CH, _D), jnp.float32),
        pltpu.VMEM((2, _CH, _D), jnp.float32),
        pltpu.VMEM((_NCHUNK, _CH), jnp.float32),
        pltpu.VMEM((1, _D), jnp.float32),
        pltpu.VMEM((1,), jnp.float32),
        pltpu.VMEM((_NG * _L * _L,), jnp.float32),
    ] + [pltpu.SemaphoreType.DMA] * 9,
)


def kernel(x, y, fc_w, fc_b, emb):
    return _sc_call(x, y.astype(jnp.int32), fc_w, fc_b, emb)
